# Initial kernel scaffold; baseline (speedup 1.0000x reference)
import jax
import jax.numpy as jnp
from jax.experimental import pallas as pl


def kernel(pred, target):
    raise NotImplementedError("write your pallas kernel here")


# speedup vs baseline: 152.6292x; 152.6292x over previous
import functools

import jax
import jax.numpy as jnp
from jax.experimental import pallas as pl
from jax.experimental.pallas import tpu as pltpu

_NUM_CLASSES = 21
_SMOOTHING = 0.1
_IGNORE_INDEX = 255
_CONFIDENCE = 1.0 - _SMOOTHING
_OFF = _SMOOTHING / (_NUM_CLASSES - 1)
import math as _math
_K_CONST = _CONFIDENCE * _math.log(_CONFIDENCE) + (_NUM_CLASSES - 1) * _OFF * _math.log(_OFF)


def _loss_body(pred_ref, tgt_ref, out_ref, acc_ref, *, nb, nh):
    b = pl.program_id(0)
    h = pl.program_id(1)

    @pl.when(jnp.logical_and(b == 0, h == 0))
    def _init():
        acc_ref[0] = 0.0
        acc_ref[1] = 0.0
        acc_ref[2] = 0.0
        acc_ref[3] = 0.0

    p = pred_ref[0]
    t = tgt_ref[0]

    m = jnp.max(p, axis=0)
    s = jnp.sum(jnp.exp(p - m[None]), axis=0)
    lse = m + jnp.log(s)
    tot = jnp.sum(p, axis=0)
    cls = jax.lax.broadcasted_iota(jnp.int32, p.shape, 0)
    pt = jnp.sum(jnp.where(cls == t[None], p, 0.0), axis=0)

    vf = (t != _IGNORE_INDEX).astype(jnp.float32)
    acc_ref[0] += jnp.sum(lse * vf)
    acc_ref[1] += jnp.sum(tot * vf)
    acc_ref[2] += jnp.sum(pt * vf)
    acc_ref[3] += jnp.sum(vf)

    @pl.when(jnp.logical_and(b == nb - 1, h == nh - 1))
    def _fini():
        count = acc_ref[3]
        total = (_K_CONST * count + acc_ref[0]
                 - _OFF * acc_ref[1]
                 - (_CONFIDENCE - _OFF) * acc_ref[2])
        loss = total / jnp.maximum(count, 1.0)
        out_ref[0, 0] = jnp.where(count > 0.0, loss, 0.0)


def kernel(pred, target):
    B, C, H, W = pred.shape
    Ht = 128 if H % 128 == 0 else H
    nh = H // Ht
    grid = (B, nh)
    out = pl.pallas_call(
        functools.partial(_loss_body, nb=B, nh=nh),
        grid=grid,
        in_specs=[
            pl.BlockSpec((1, C, Ht, W), lambda b, h: (b, 0, h, 0)),
            pl.BlockSpec((1, Ht, W), lambda b, h: (b, h, 0)),
        ],
        out_specs=pl.BlockSpec(memory_space=pltpu.SMEM),
        out_shape=jax.ShapeDtypeStruct((1, 1), jnp.float32),
        scratch_shapes=[pltpu.SMEM((4,), jnp.float32)],
        compiler_params=pltpu.CompilerParams(
            dimension_semantics=("arbitrary", "arbitrary"),
        ),
    )(pred, target)
    return out[0, 0]
